# baseline (speedup 1.0000x reference)
import functools

import jax
import jax.numpy as jnp
from jax import lax
from jax.experimental import pallas as pl
from jax.experimental.pallas import tpu as pltpu
from jax.experimental.pallas import tpu_sc as plsc

H, W = 480, 640
HW = H * W
N_EV = 500000
FLOW_SCALING = 128.0

NUM_WORKERS = 32
BPW = 15680
NSUB = 2
SUB = BPW // NSUB
NPAD = NUM_WORKERS * BPW


def _sc_gather_body(el_hbm, flow_hbm, out_hbm, el_v, ii_v, vals_v, sem):
    wid = lax.axis_index("s") * 2 + lax.axis_index("c")
    base = wid * BPW
    lane = lax.iota(jnp.int32, 16)
    sp4 = (lane >> 1) * 4
    xoff = sp4 + 1
    yoff = sp4 + 2
    par = (lane & 1) * HW

    def sub(j, carry):
        off = base + j * SUB
        pltpu.sync_copy(el_hbm.at[pl.ds(off * 4, SUB * 4)], el_v)

        def grp(g, c2):
            o = g * 32
            xs = plsc.load_gather(el_v, [o + xoff])
            ys = plsc.load_gather(el_v, [o + yoff])
            ii_v[pl.ds(g * 16, 16)] = xs + ys * W + par
            return c2

        lax.fori_loop(0, SUB // 8, grp, 0)
        pltpu.async_copy(flow_hbm.at[ii_v], vals_v, sem).wait()
        pltpu.sync_copy(vals_v, out_hbm.at[pl.ds(off * 2, SUB * 2)])
        return carry

    lax.fori_loop(0, NSUB, sub, 0)


@functools.lru_cache(maxsize=1)
def _sc_gather():
    return functools.partial(
        pl.kernel,
        out_type=jax.ShapeDtypeStruct((NPAD * 2,), jnp.float32),
        mesh=plsc.VectorSubcoreMesh(core_axis_name="c", subcore_axis_name="s"),
        compiler_params=pltpu.CompilerParams(
            needs_layout_passes=False, use_tc_tiling_on_sc=False),
        scratch_types=[
            pltpu.VMEM((SUB * 4,), jnp.int32),
            pltpu.VMEM((SUB * 2,), jnp.int32),
            pltpu.VMEM((SUB * 2,), jnp.float32),
            pltpu.SemaphoreType.DMA,
        ],
    )(_sc_gather_body)


def _warp_body(flow_ref, out_ref):
    f = flow_ref[...]
    xii = lax.broadcasted_iota(jnp.int32, (1, H, W), 2)
    yii = lax.broadcasted_iota(jnp.int32, (1, H, W), 1)
    xi = xii.astype(jnp.float32)
    yi = yii.astype(jnp.float32)
    gx = 2.0 * xi / (W - 1) - 1.0
    gy = 2.0 * yi / (H - 1) - 1.0
    px = (gx + 1.0) * (W - 1) / 2.0
    py = (gy + 1.0) * (H - 1) / 2.0
    x0 = jnp.floor(px)
    y0 = jnp.floor(py)
    wx = px - x0
    wy = py - y0
    x0i = jnp.clip(x0.astype(jnp.int32), 0, W - 1)
    x1i = jnp.clip(x0i + 1, 0, W - 1)
    y0i = jnp.clip(y0.astype(jnp.int32), 0, H - 1)
    y1i = jnp.clip(y0i + 1, 0, H - 1)
    fxm = jnp.concatenate([f[:, :, :1], f[:, :, :-1]], axis=2)
    fxp = jnp.concatenate([f[:, :, 1:], f[:, :, -1:]], axis=2)
    g0 = jnp.where(x0i == xii, f, fxm)
    g1 = jnp.where(x1i == xii, f, fxp)
    g0u = jnp.concatenate([g0[:, :1, :], g0[:, :-1, :]], axis=1)
    g0d = jnp.concatenate([g0[:, 1:, :], g0[:, -1:, :]], axis=1)
    g1u = jnp.concatenate([g1[:, :1, :], g1[:, :-1, :]], axis=1)
    g1d = jnp.concatenate([g1[:, 1:, :], g1[:, -1:, :]], axis=1)
    cy0 = y0i == yii
    cy1 = y1i == yii
    v00 = jnp.where(cy0, g0, g0u)
    v01 = jnp.where(cy0, g1, g1u)
    v10 = jnp.where(cy1, g0, g0d)
    v11 = jnp.where(cy1, g1, g1d)
    samp = (v00 * (1.0 - wy) * (1.0 - wx) + v01 * (1.0 - wy) * wx
            + v10 * wy * (1.0 - wx) + v11 * wy * wx)
    ind = jnp.concatenate([xi, yi], axis=0)
    warped = ind + samp * FLOW_SCALING
    out_ref[...] = warped - ind


_warp = pl.pallas_call(
    _warp_body,
    out_shape=jax.ShapeDtypeStruct((2, H, W), jnp.float32),
)


def kernel(flow, event_list, event_mask, dt_input, dt_gt):
    flow_flat = flow.reshape(2 * HW)
    el = event_list.reshape(N_EV, 4)
    el_pad = jnp.concatenate(
        [el, jnp.zeros((NPAD - N_EV, 4), jnp.int32)], axis=0).reshape(-1)
    pairs = _sc_gather()(el_pad, flow_flat)
    event_flow = pairs[:N_EV * 2].reshape(1, N_EV, 2)
    accum = _warp(flow.reshape(2, H, W)).reshape(1, 2, H, W)
    return event_flow, accum


# speedup vs baseline: 1.2212x; 1.2212x over previous
import functools

import jax
import jax.numpy as jnp
from jax import lax
from jax.experimental import pallas as pl
from jax.experimental.pallas import tpu as pltpu
from jax.experimental.pallas import tpu_sc as plsc

H, W = 480, 640
HW = H * W
N_EV = 500000
FLOW_SCALING = 128.0

NUM_WORKERS = 32
BPW = 15616
NSUB = 2
SUB = BPW // NSUB
MAIN = NUM_WORKERS * BPW
TAIL = N_EV - MAIN


def _sc_gather_body(el_hbm, flow_hbm, out_hbm, el_v, ii_v, vals_v, sem):
    wid = lax.axis_index("s") * 2 + lax.axis_index("c")
    base = wid * BPW
    lane = lax.iota(jnp.int32, 16)
    sp4 = (lane >> 1) * 4
    xoff = sp4 + 1
    yoff = sp4 + 2
    par = (lane & 1) * HW

    def do_chunk(off, elr, iir, valr, ngrp):
        pltpu.sync_copy(el_hbm.at[pl.ds(off * 4, ngrp * 32)], elr)

        def grp(g, c2):
            o = g * 32
            xs = plsc.load_gather(elr, [o + xoff])
            ys = plsc.load_gather(elr, [o + yoff])
            iir[pl.ds(g * 16, 16)] = xs + ys * W + par
            return c2

        lax.fori_loop(0, ngrp, grp, 0)
        pltpu.async_copy(flow_hbm.at[iir], valr, sem).wait()
        pltpu.sync_copy(valr, out_hbm.at[pl.ds(off * 2, ngrp * 16)])

    def sub(j, carry):
        do_chunk(base + j * SUB, el_v, ii_v, vals_v, SUB // 8)
        return carry

    lax.fori_loop(0, NSUB, sub, 0)

    @pl.when(wid == NUM_WORKERS - 1)
    def _tail():
        do_chunk(MAIN,
                 el_v.at[pl.ds(0, TAIL * 4)],
                 ii_v.at[pl.ds(0, TAIL * 2)],
                 vals_v.at[pl.ds(0, TAIL * 2)],
                 TAIL // 8)


@functools.lru_cache(maxsize=1)
def _sc_gather():
    return functools.partial(
        pl.kernel,
        out_type=jax.ShapeDtypeStruct((N_EV * 2,), jnp.float32),
        mesh=plsc.VectorSubcoreMesh(core_axis_name="c", subcore_axis_name="s"),
        compiler_params=pltpu.CompilerParams(
            needs_layout_passes=False, use_tc_tiling_on_sc=False),
        scratch_types=[
            pltpu.VMEM((SUB * 4,), jnp.int32),
            pltpu.VMEM((SUB * 2,), jnp.int32),
            pltpu.VMEM((SUB * 2,), jnp.float32),
            pltpu.SemaphoreType.DMA,
        ],
    )(_sc_gather_body)


def _warp_body(flow_ref, out_ref):
    f = flow_ref[...]
    xii = lax.broadcasted_iota(jnp.int32, (1, H, W), 2)
    yii = lax.broadcasted_iota(jnp.int32, (1, H, W), 1)
    xi = xii.astype(jnp.float32)
    yi = yii.astype(jnp.float32)
    gx = 2.0 * xi / (W - 1) - 1.0
    gy = 2.0 * yi / (H - 1) - 1.0
    px = (gx + 1.0) * (W - 1) / 2.0
    py = (gy + 1.0) * (H - 1) / 2.0
    x0 = jnp.floor(px)
    y0 = jnp.floor(py)
    wx = px - x0
    wy = py - y0
    x0i = jnp.clip(x0.astype(jnp.int32), 0, W - 1)
    x1i = jnp.clip(x0i + 1, 0, W - 1)
    y0i = jnp.clip(y0.astype(jnp.int32), 0, H - 1)
    y1i = jnp.clip(y0i + 1, 0, H - 1)
    fxm = jnp.concatenate([f[:, :, :1], f[:, :, :-1]], axis=2)
    fxp = jnp.concatenate([f[:, :, 1:], f[:, :, -1:]], axis=2)
    g0 = jnp.where(x0i == xii, f, fxm)
    g1 = jnp.where(x1i == xii, f, fxp)
    g0u = jnp.concatenate([g0[:, :1, :], g0[:, :-1, :]], axis=1)
    g0d = jnp.concatenate([g0[:, 1:, :], g0[:, -1:, :]], axis=1)
    g1u = jnp.concatenate([g1[:, :1, :], g1[:, :-1, :]], axis=1)
    g1d = jnp.concatenate([g1[:, 1:, :], g1[:, -1:, :]], axis=1)
    cy0 = y0i == yii
    cy1 = y1i == yii
    v00 = jnp.where(cy0, g0, g0u)
    v01 = jnp.where(cy0, g1, g1u)
    v10 = jnp.where(cy1, g0, g0d)
    v11 = jnp.where(cy1, g1, g1d)
    samp = (v00 * (1.0 - wy) * (1.0 - wx) + v01 * (1.0 - wy) * wx
            + v10 * wy * (1.0 - wx) + v11 * wy * wx)
    ind = jnp.concatenate([xi, yi], axis=0)
    warped = ind + samp * FLOW_SCALING
    out_ref[...] = warped - ind


_warp = pl.pallas_call(
    _warp_body,
    out_shape=jax.ShapeDtypeStruct((2, H, W), jnp.float32),
)


def kernel(flow, event_list, event_mask, dt_input, dt_gt):
    flow_flat = flow.reshape(2 * HW)
    el_flat = event_list.reshape(N_EV * 4)
    pairs = _sc_gather()(el_flat, flow_flat)
    event_flow = pairs.reshape(1, N_EV, 2)
    accum = _warp(flow.reshape(2, H, W)).reshape(1, 2, H, W)
    return event_flow, accum


# speedup vs baseline: 12.3117x; 10.0814x over previous
import functools

import jax
import jax.numpy as jnp
from jax import lax
from jax.experimental import pallas as pl
from jax.experimental.pallas import tpu as pltpu
from jax.experimental.pallas import tpu_sc as plsc

H, W = 480, 640
HW = H * W
N_EV = 500000
FLOW_SCALING = 128.0

NUM_WORKERS = 32
BPW = 15616
MAIN = NUM_WORKERS * BPW
TAIL = N_EV - MAIN


def _sc_gather_body(xs_hbm, ys_hbm, flow_hbm, outx_hbm, outy_hbm,
                    xs_v, ys_v, iix_v, iiy_v, valx_v, valy_v, sem):
    wid = lax.axis_index("s") * 2 + lax.axis_index("c")
    base = wid * BPW

    def do_chunk(off, xsr, ysr, iixr, iiyr, vxr, vyr, ngrp):
        n = ngrp * 16
        pltpu.sync_copy(xs_hbm.at[pl.ds(off, n)], xsr)
        pltpu.sync_copy(ys_hbm.at[pl.ds(off, n)], ysr)

        def grp(g, c2):
            s = pl.ds(g * 16, 16)
            ii = xsr[s] + ysr[s] * W
            iixr[s] = ii
            iiyr[s] = ii + HW
            return c2

        lax.fori_loop(0, ngrp, grp, 0)
        cx = pltpu.async_copy(flow_hbm.at[iixr], vxr, sem)
        cy = pltpu.async_copy(flow_hbm.at[iiyr], vyr, sem)
        cx.wait()
        cy.wait()
        pltpu.sync_copy(vxr, outx_hbm.at[pl.ds(off, n)])
        pltpu.sync_copy(vyr, outy_hbm.at[pl.ds(off, n)])

    do_chunk(base, xs_v, ys_v, iix_v, iiy_v, valx_v, valy_v, BPW // 16)

    @pl.when(wid == NUM_WORKERS - 1)
    def _tail():
        t = TAIL
        do_chunk(MAIN,
                 xs_v.at[pl.ds(0, t)], ys_v.at[pl.ds(0, t)],
                 iix_v.at[pl.ds(0, t)], iiy_v.at[pl.ds(0, t)],
                 valx_v.at[pl.ds(0, t)], valy_v.at[pl.ds(0, t)],
                 t // 16)


@functools.lru_cache(maxsize=1)
def _sc_gather():
    return functools.partial(
        pl.kernel,
        out_type=(jax.ShapeDtypeStruct((N_EV,), jnp.float32),
                  jax.ShapeDtypeStruct((N_EV,), jnp.float32)),
        mesh=plsc.VectorSubcoreMesh(core_axis_name="c", subcore_axis_name="s"),
        compiler_params=pltpu.CompilerParams(
            needs_layout_passes=False, use_tc_tiling_on_sc=False),
        scratch_types=[
            pltpu.VMEM((BPW,), jnp.int32),
            pltpu.VMEM((BPW,), jnp.int32),
            pltpu.VMEM((BPW,), jnp.int32),
            pltpu.VMEM((BPW,), jnp.int32),
            pltpu.VMEM((BPW,), jnp.float32),
            pltpu.VMEM((BPW,), jnp.float32),
            pltpu.SemaphoreType.DMA,
        ],
    )(_sc_gather_body)


def _warp_body(flow_ref, out_ref):
    f = flow_ref[...]
    xii = lax.broadcasted_iota(jnp.int32, (1, H, W), 2)
    yii = lax.broadcasted_iota(jnp.int32, (1, H, W), 1)
    xi = xii.astype(jnp.float32)
    yi = yii.astype(jnp.float32)
    gx = 2.0 * xi / (W - 1) - 1.0
    gy = 2.0 * yi / (H - 1) - 1.0
    px = (gx + 1.0) * (W - 1) / 2.0
    py = (gy + 1.0) * (H - 1) / 2.0
    x0 = jnp.floor(px)
    y0 = jnp.floor(py)
    wx = px - x0
    wy = py - y0
    x0i = jnp.clip(x0.astype(jnp.int32), 0, W - 1)
    x1i = jnp.clip(x0i + 1, 0, W - 1)
    y0i = jnp.clip(y0.astype(jnp.int32), 0, H - 1)
    y1i = jnp.clip(y0i + 1, 0, H - 1)
    fxm = jnp.concatenate([f[:, :, :1], f[:, :, :-1]], axis=2)
    fxp = jnp.concatenate([f[:, :, 1:], f[:, :, -1:]], axis=2)
    g0 = jnp.where(x0i == xii, f, fxm)
    g1 = jnp.where(x1i == xii, f, fxp)
    g0u = jnp.concatenate([g0[:, :1, :], g0[:, :-1, :]], axis=1)
    g0d = jnp.concatenate([g0[:, 1:, :], g0[:, -1:, :]], axis=1)
    g1u = jnp.concatenate([g1[:, :1, :], g1[:, :-1, :]], axis=1)
    g1d = jnp.concatenate([g1[:, 1:, :], g1[:, -1:, :]], axis=1)
    cy0 = y0i == yii
    cy1 = y1i == yii
    v00 = jnp.where(cy0, g0, g0u)
    v01 = jnp.where(cy0, g1, g1u)
    v10 = jnp.where(cy1, g0, g0d)
    v11 = jnp.where(cy1, g1, g1d)
    samp = (v00 * (1.0 - wy) * (1.0 - wx) + v01 * (1.0 - wy) * wx
            + v10 * wy * (1.0 - wx) + v11 * wy * wx)
    ind = jnp.concatenate([xi, yi], axis=0)
    warped = ind + samp * FLOW_SCALING
    out_ref[...] = warped - ind


_warp = pl.pallas_call(
    _warp_body,
    out_shape=jax.ShapeDtypeStruct((2, H, W), jnp.float32),
)


def kernel(flow, event_list, event_mask, dt_input, dt_gt):
    flow_flat = flow.reshape(2 * HW)
    xs = event_list[0, :, 1]
    ys = event_list[0, :, 2]
    xf, yf = _sc_gather()(xs, ys, flow_flat)
    event_flow = jnp.stack([xf, yf], axis=-1)[None]
    accum = _warp(flow.reshape(2, H, W)).reshape(1, 2, H, W)
    return event_flow, accum


# speedup vs baseline: 14.5024x; 1.1779x over previous
import functools

import jax
import jax.numpy as jnp
from jax import lax
from jax.experimental import pallas as pl
from jax.experimental.pallas import tpu as pltpu
from jax.experimental.pallas import tpu_sc as plsc

H, W = 480, 640
HW = H * W
N_EV = 500000
FLOW_SCALING = 128.0

NUM_WORKERS = 32
BPW = 15616
MAIN = NUM_WORKERS * BPW
TAIL = N_EV - MAIN


def _sc_gather_body(xs_hbm, ys_hbm, flow_hbm, out_hbm,
                    xs_v, ys_v, iix_v, iiy_v, valx_v, valy_v, sem):
    wid = lax.axis_index("s") * 2 + lax.axis_index("c")
    base = wid * BPW

    def do_chunk(off, xsr, ysr, iixr, iiyr, vxr, vyr, ngrp):
        n = ngrp * 16
        pltpu.sync_copy(xs_hbm.at[pl.ds(off, n)], xsr)
        pltpu.sync_copy(ys_hbm.at[pl.ds(off, n)], ysr)

        def grp(g, c2):
            s = pl.ds(g * 16, 16)
            ii = xsr[s] + ysr[s] * W
            iixr[s] = ii
            iiyr[s] = ii + HW
            return c2

        lax.fori_loop(0, ngrp, grp, 0)
        cx = pltpu.async_copy(flow_hbm.at[iixr], vxr, sem)
        cy = pltpu.async_copy(flow_hbm.at[iiyr], vyr, sem)
        cx.wait()
        cy.wait()
        pltpu.sync_copy(vxr, out_hbm.at[pl.ds(off, n)])
        pltpu.sync_copy(vyr, out_hbm.at[pl.ds(N_EV + off, n)])

    do_chunk(base, xs_v, ys_v, iix_v, iiy_v, valx_v, valy_v, BPW // 16)

    @pl.when(wid == NUM_WORKERS - 1)
    def _tail():
        t = TAIL
        do_chunk(MAIN,
                 xs_v.at[pl.ds(0, t)], ys_v.at[pl.ds(0, t)],
                 iix_v.at[pl.ds(0, t)], iiy_v.at[pl.ds(0, t)],
                 valx_v.at[pl.ds(0, t)], valy_v.at[pl.ds(0, t)],
                 t // 16)


@functools.lru_cache(maxsize=1)
def _sc_gather():
    return functools.partial(
        pl.kernel,
        out_type=jax.ShapeDtypeStruct((2 * N_EV,), jnp.float32),
        mesh=plsc.VectorSubcoreMesh(core_axis_name="c", subcore_axis_name="s"),
        compiler_params=pltpu.CompilerParams(
            needs_layout_passes=False, use_tc_tiling_on_sc=False),
        scratch_types=[
            pltpu.VMEM((BPW,), jnp.int32),
            pltpu.VMEM((BPW,), jnp.int32),
            pltpu.VMEM((BPW,), jnp.int32),
            pltpu.VMEM((BPW,), jnp.int32),
            pltpu.VMEM((BPW,), jnp.float32),
            pltpu.VMEM((BPW,), jnp.float32),
            pltpu.SemaphoreType.DMA,
        ],
    )(_sc_gather_body)


def _warp_body(flow_ref, out_ref):
    f = flow_ref[...]
    xii = lax.broadcasted_iota(jnp.int32, (1, H, W), 2)
    yii = lax.broadcasted_iota(jnp.int32, (1, H, W), 1)
    xi = xii.astype(jnp.float32)
    yi = yii.astype(jnp.float32)
    gx = 2.0 * xi / (W - 1) - 1.0
    gy = 2.0 * yi / (H - 1) - 1.0
    px = (gx + 1.0) * (W - 1) / 2.0
    py = (gy + 1.0) * (H - 1) / 2.0
    x0 = jnp.floor(px)
    y0 = jnp.floor(py)
    wx = px - x0
    wy = py - y0
    x0i = jnp.clip(x0.astype(jnp.int32), 0, W - 1)
    x1i = jnp.clip(x0i + 1, 0, W - 1)
    y0i = jnp.clip(y0.astype(jnp.int32), 0, H - 1)
    y1i = jnp.clip(y0i + 1, 0, H - 1)
    fxm = jnp.concatenate([f[:, :, :1], f[:, :, :-1]], axis=2)
    fxp = jnp.concatenate([f[:, :, 1:], f[:, :, -1:]], axis=2)
    g0 = jnp.where(x0i == xii, f, fxm)
    g1 = jnp.where(x1i == xii, f, fxp)
    g0u = jnp.concatenate([g0[:, :1, :], g0[:, :-1, :]], axis=1)
    g0d = jnp.concatenate([g0[:, 1:, :], g0[:, -1:, :]], axis=1)
    g1u = jnp.concatenate([g1[:, :1, :], g1[:, :-1, :]], axis=1)
    g1d = jnp.concatenate([g1[:, 1:, :], g1[:, -1:, :]], axis=1)
    cy0 = y0i == yii
    cy1 = y1i == yii
    v00 = jnp.where(cy0, g0, g0u)
    v01 = jnp.where(cy0, g1, g1u)
    v10 = jnp.where(cy1, g0, g0d)
    v11 = jnp.where(cy1, g1, g1d)
    samp = (v00 * (1.0 - wy) * (1.0 - wx) + v01 * (1.0 - wy) * wx
            + v10 * wy * (1.0 - wx) + v11 * wy * wx)
    ind = jnp.concatenate([xi, yi], axis=0)
    warped = ind + samp * FLOW_SCALING
    out_ref[...] = warped - ind


_warp = pl.pallas_call(
    _warp_body,
    out_shape=jax.ShapeDtypeStruct((2, H, W), jnp.float32),
)


def kernel(flow, event_list, event_mask, dt_input, dt_gt):
    flow_flat = flow.reshape(2 * HW)
    xs = event_list[0, :, 1]
    ys = event_list[0, :, 2]
    planes = _sc_gather()(xs, ys, flow_flat)
    event_flow = planes.reshape(2, N_EV).T[None]
    accum = _warp(flow.reshape(2, H, W)).reshape(1, 2, H, W)
    return event_flow, accum
